# baseline (speedup 1.0000x reference)
import math

import jax
import jax.numpy as jnp
from jax import lax
from jax.experimental import pallas as pl
from jax.experimental.pallas import tpu as pltpu
from jax.experimental.pallas import tpu_sc as plsc

_VOCAB = 1000000
_D = 64
_BATCH = 4096
_HIST = 200
_B = _BATCH * _HIST
_SCALE = math.sqrt(_D)

_NC = 2
_NS = 16
_NW = _NC * _NS
_BPW = _B // _NW
_C = 512
_NCHUNK = _BPW // _C
_LANES = 16
_VPR = _D // _LANES


def _sc_body(table_hbm, idx_hbm, out_hbm, idx0, idx1, rows0, rows1, gsem, osem):
    wid = lax.axis_index("s") * _NC + lax.axis_index("c")
    base = wid * _BPW

    rows = (rows0, rows1)
    idxs = (idx0, idx1)

    @pl.loop(0, _NCHUNK, step=2)
    def _chunks(g0):
        for b in range(2):
            g = g0 + b
            buf = rows[b]
            ibuf = idxs[b]
            pltpu.sync_copy(idx_hbm.at[wid, g], ibuf)
            pltpu.async_copy(table_hbm.at[ibuf], buf, gsem).wait()

            @pl.loop(0, _C)
            def _scale(r):
                for j in range(_VPR):
                    sl = pl.ds(j * _LANES, _LANES)
                    buf[r, sl] = buf[r, sl] * _SCALE

            pltpu.async_copy(
                buf, out_hbm.at[pl.ds(base + g * _C, _C)], osem
            ).wait()


def _embed_gather(table, idx2d):
    mesh = plsc.VectorSubcoreMesh(core_axis_name="c", subcore_axis_name="s")
    f = pl.kernel(
        _sc_body,
        out_type=jax.ShapeDtypeStruct((_B, _D), jnp.float32),
        mesh=mesh,
        scratch_types=[
            pltpu.VMEM((_C,), jnp.int32),
            pltpu.VMEM((_C,), jnp.int32),
            pltpu.VMEM((_C, _D), jnp.float32),
            pltpu.VMEM((_C, _D), jnp.float32),
            pltpu.SemaphoreType.DMA,
            pltpu.SemaphoreType.DMA,
        ],
        compiler_params=pltpu.CompilerParams(use_tc_tiling_on_sc=False),
    )
    return f(table, idx2d)


def kernel(x, table):
    idx = x.reshape(_NW, _NCHUNK, _C).astype(jnp.int32)
    out = _embed_gather(table, idx)
    return out.reshape(_BATCH, _HIST, _D)


# speedup vs baseline: 1.1321x; 1.1321x over previous
import math

import jax
import jax.numpy as jnp
from jax import lax
from jax.experimental import pallas as pl
from jax.experimental.pallas import tpu as pltpu
from jax.experimental.pallas import tpu_sc as plsc

_VOCAB = 1000000
_D = 64
_BATCH = 4096
_HIST = 200
_B = _BATCH * _HIST
_SCALE = math.sqrt(_D)

_NC = 2
_NS = 16
_NW = _NC * _NS
_BPW = _B // _NW
_C = 400
_NCHUNK = _BPW // _C
_LANES = 16
_VPR = _D // _LANES


def _sc_body(table_hbm, idx_hbm, out_hbm,
             idx_all, a0, a1, b0, b1, gs0, gs1, os0, os1):
    wid = lax.axis_index("s") * _NC + lax.axis_index("c")
    base = wid * _BPW

    A = (a0, a1)
    Bb = (b0, b1)
    gsem = (gs0, gs1)
    osem = (os0, os1)

    pltpu.sync_copy(idx_hbm.at[wid], idx_all)

    def start_gather(b, g):
        pltpu.async_copy(table_hbm.at[idx_all.at[g]], A[b], gsem[b])

    def wait_gather(b):
        pltpu.make_async_copy(
            table_hbm.at[pl.ds(0, _C)], A[b], gsem[b]
        ).wait()

    def scale(b):
        src = A[b]
        dst = Bb[b]

        @pl.loop(0, _C, step=8)
        def _rows(r0):
            for dr in range(8):
                r = r0 + dr
                for j in range(_VPR):
                    sl = pl.ds(j * _LANES, _LANES)
                    dst[r, sl] = src[r, sl] * _SCALE

    def start_write(b, g):
        pltpu.async_copy(
            Bb[b], out_hbm.at[pl.ds(base + g * _C, _C)], osem[b]
        )

    def wait_write(b):
        pltpu.make_async_copy(
            Bb[b], out_hbm.at[pl.ds(base, _C)], osem[b]
        ).wait()

    for b in range(2):
        start_gather(b, b)
    for b in range(2):
        wait_gather(b)
        scale(b)
        start_write(b, b)
        start_gather(b, b + 2)

    @pl.loop(2, _NCHUNK - 2, step=2)
    def _main(g0):
        for b in range(2):
            g = g0 + b
            wait_gather(b)
            wait_write(b)
            scale(b)
            start_write(b, g)
            start_gather(b, g + 2)

    for b in range(2):
        g = _NCHUNK - 2 + b
        wait_gather(b)
        wait_write(b)
        scale(b)
        start_write(b, g)
    for b in range(2):
        wait_write(b)


def _embed_gather(table, idx3d):
    mesh = plsc.VectorSubcoreMesh(core_axis_name="c", subcore_axis_name="s")
    f = pl.kernel(
        _sc_body,
        out_type=jax.ShapeDtypeStruct((_B, _D), jnp.float32),
        mesh=mesh,
        scratch_types=[
            pltpu.VMEM((_NCHUNK, _C), jnp.int32),
            pltpu.VMEM((_C, _D), jnp.float32),
            pltpu.VMEM((_C, _D), jnp.float32),
            pltpu.VMEM((_C, _D), jnp.float32),
            pltpu.VMEM((_C, _D), jnp.float32),
            pltpu.SemaphoreType.DMA,
            pltpu.SemaphoreType.DMA,
            pltpu.SemaphoreType.DMA,
            pltpu.SemaphoreType.DMA,
        ],
        compiler_params=pltpu.CompilerParams(use_tc_tiling_on_sc=False),
    )
    return f(table, idx3d)


def kernel(x, table):
    idx = x.reshape(_NW, _NCHUNK, _C).astype(jnp.int32)
    out = _embed_gather(table, idx)
    return out.reshape(_BATCH, _HIST, _D)
